# Initial kernel scaffold; baseline (speedup 1.0000x reference)
import functools

import jax
import jax.numpy as jnp
from jax import lax
from jax.experimental import pallas as pl
from jax.experimental.pallas import tpu as pltpu
from jax.experimental.pallas import tpu_sc as plsc

N = 10000
E = 320000
D = 128
DE = 16
DH = 128
NOUT = 128
EOUT = 16

NC = 2
NS = 16
NW = NC * NS
EPW = E // NW
C1 = 80
NCH1 = EPW // C1
C2 = 80
NCH2 = EPW // C2
RPT = N // NS
W1 = DH + 16

_HIGH = jax.lax.Precision.HIGHEST


def _tc1a_body(xs_ref, xd_ref, wms_ref, wmd_ref, wes_ref, bm_ref,
               a_ref, b_ref, p_ref):
    xs = xs_ref[...]
    xd = xd_ref[...]
    a_ref[...] = jnp.dot(xs, wms_ref[...], precision=_HIGH) + bm_ref[...]
    b_ref[...] = jnp.dot(xd, wmd_ref[...], precision=_HIGH)
    p_ref[...] = jnp.dot(xs, wes_ref[...], precision=_HIGH)


def _tc1b_body(e_ref, wme_ref, wee_ref, be_ref, ce_ref, re_ref):
    ev = e_ref[...]
    ce_ref[...] = jnp.dot(ev, wme_ref[...], precision=_HIGH)
    re_ref[...] = jnp.dot(ev, wee_ref[...], precision=_HIGH) + be_ref[...]


def _tc2_body(part_ref, xd_ref, weh_ref, wad_ref, wah_ref, ba_ref,
              q_ref, g_ref, invd_ref):
    hs = part_ref[0, :, 0:DH] + part_ref[1, :, 0:DH]
    deg = part_ref[0, :, DH:DH + 1] + part_ref[1, :, DH:DH + 1]
    denom = jnp.maximum(deg, 1.0)
    hn = hs / denom
    mask = (deg > 0.0).astype(jnp.float32)
    q_ref[...] = jnp.dot(hn, weh_ref[...], precision=_HIGH)
    g_ref[...] = mask * (jnp.dot(xd_ref[...], wad_ref[...], precision=_HIGH)
                         + jnp.dot(hn, wah_ref[...], precision=_HIGH)
                         + ba_ref[...])
    invd_ref[...] = jnp.broadcast_to(1.0 / denom, invd_ref.shape)


def _tc3_body(sp_ref, g_ref, invd_ref, wae_ref, out_ref):
    s = sp_ref[0] + sp_ref[1]
    out_ref[...] = g_ref[...] + (jnp.dot(s, wae_ref[...], precision=_HIGH)
                                 * invd_ref[:, 0:1])


def _node_block(nb):
    return pl.BlockSpec((nb, None), lambda i: (i, 0))


def _full_block():
    return pl.BlockSpec(lambda i: (0, 0))


def _sc1_body(a_hbm, b_hbm, ce_hbm, src_hbm, dst_hbm, out_hbm,
              sidx, didx, abuf, bbuf, cbuf, mbuf, acc):
    cid = lax.axis_index("c")
    sid = lax.axis_index("s")
    wid = sid * NC + cid

    def zrow(r, _):
        for g in range(W1 // 16):
            mbuf[r, pl.ds(g * 16, 16)] = jnp.zeros((16,), jnp.float32)
        return 0
    lax.fori_loop(0, C1, zrow, 0)
    base_r = sid * RPT
    for j in range(RPT // C1):
        pltpu.sync_copy(mbuf, acc.at[pl.ds(base_r + j * C1, C1)])
    rem = RPT - (RPT // C1) * C1
    pltpu.sync_copy(mbuf.at[pl.ds(0, rem)],
                    acc.at[pl.ds(base_r + (RPT // C1) * C1, rem)])

    onehot = jnp.where(lax.iota(jnp.int32, 16) == 0,
                       jnp.float32(1.0), jnp.float32(0.0))

    def setc(r, _):
        mbuf[r, pl.ds(DH, 16)] = onehot
        return 0
    lax.fori_loop(0, C1, setc, 0)

    plsc.subcore_barrier()

    def chunk(k, _):
        base = wid * EPW + k * C1
        pltpu.sync_copy(src_hbm.at[pl.ds(base, C1)], sidx)
        pltpu.sync_copy(dst_hbm.at[pl.ds(base, C1)], didx)
        pltpu.sync_copy(a_hbm.at[sidx], abuf)
        pltpu.sync_copy(b_hbm.at[didx], bbuf)
        pltpu.sync_copy(ce_hbm.at[pl.ds(base, C1)], cbuf)

        def row(r, _):
            for g in range(DH // 16):
                sl = pl.ds(g * 16, 16)
                v = abuf[r, sl] + bbuf[r, sl] + cbuf[r, sl]
                mbuf[r, sl] = jnp.maximum(v, jnp.float32(0.0))
            return 0
        lax.fori_loop(0, C1, row, 0)

        pltpu.sync_copy(mbuf, acc.at[didx], add=True)
        return 0

    lax.fori_loop(0, NCH1, chunk, 0)
    plsc.subcore_barrier()

    pltpu.sync_copy(acc.at[pl.ds(base_r, RPT)],
                    out_hbm.at[cid, pl.ds(base_r, RPT)])


def _sc2_body(p_hbm, q_hbm, re_hbm, src_hbm, dst_hbm, eh_hbm, out_hbm,
              sidx, didx, pbuf, qbuf, rbuf, ebuf, acc):
    cid = lax.axis_index("c")
    sid = lax.axis_index("s")
    wid = sid * NC + cid

    def zrow(r, _):
        ebuf[r, pl.ds(0, 16)] = jnp.zeros((16,), jnp.float32)
        return 0
    lax.fori_loop(0, C2, zrow, 0)
    base_r = sid * RPT
    for j in range(RPT // C2):
        pltpu.sync_copy(ebuf, acc.at[pl.ds(base_r + j * C2, C2)])
    rem = RPT - (RPT // C2) * C2
    pltpu.sync_copy(ebuf.at[pl.ds(0, rem)],
                    acc.at[pl.ds(base_r + (RPT // C2) * C2, rem)])
    plsc.subcore_barrier()

    def chunk(k, _):
        base = wid * EPW + k * C2
        pltpu.sync_copy(src_hbm.at[pl.ds(base, C2)], sidx)
        pltpu.sync_copy(dst_hbm.at[pl.ds(base, C2)], didx)
        pltpu.sync_copy(p_hbm.at[sidx], pbuf)
        pltpu.sync_copy(q_hbm.at[didx], qbuf)
        pltpu.sync_copy(re_hbm.at[pl.ds(base, C2)], rbuf)

        def row(r, _):
            sl = pl.ds(0, 16)
            v = pbuf[r, sl] + qbuf[r, sl] + rbuf[r, sl]
            ebuf[r, sl] = jnp.maximum(v, jnp.float32(0.0))
            return 0
        lax.fori_loop(0, C2, row, 0)

        pltpu.sync_copy(ebuf, eh_hbm.at[pl.ds(base, C2)])
        pltpu.sync_copy(ebuf, acc.at[didx], add=True)
        return 0

    lax.fori_loop(0, NCH2, chunk, 0)
    plsc.subcore_barrier()

    pltpu.sync_copy(acc.at[pl.ds(base_r, RPT)],
                    out_hbm.at[cid, pl.ds(base_r, RPT)])


_sc_mesh = plsc.VectorSubcoreMesh(core_axis_name="c", subcore_axis_name="s",
                                  num_cores=NC, num_subcores=NS)

_sc1 = functools.partial(
    pl.kernel, _sc1_body,
    out_type=jax.ShapeDtypeStruct((NC, N, W1), jnp.float32),
    mesh=_sc_mesh,
    scratch_types=[
        pltpu.VMEM((C1,), jnp.int32),
        pltpu.VMEM((C1,), jnp.int32),
        pltpu.VMEM((C1, DH), jnp.float32),
        pltpu.VMEM((C1, DH), jnp.float32),
        pltpu.VMEM((C1, DH), jnp.float32),
        pltpu.VMEM((C1, W1), jnp.float32),
        pltpu.VMEM_SHARED((N, W1), jnp.float32),
    ],
)()

_sc2 = functools.partial(
    pl.kernel, _sc2_body,
    out_type=[jax.ShapeDtypeStruct((E, EOUT), jnp.float32),
              jax.ShapeDtypeStruct((NC, N, EOUT), jnp.float32)],
    mesh=_sc_mesh,
    scratch_types=[
        pltpu.VMEM((C2,), jnp.int32),
        pltpu.VMEM((C2,), jnp.int32),
        pltpu.VMEM((C2, EOUT), jnp.float32),
        pltpu.VMEM((C2, EOUT), jnp.float32),
        pltpu.VMEM((C2, EOUT), jnp.float32),
        pltpu.VMEM((C2, EOUT), jnp.float32),
        pltpu.VMEM_SHARED((N, EOUT), jnp.float32),
    ],
)()


def kernel(x_src, x_dst, e, W_msg_w, W_msg_b, W_edge_w, W_edge_b,
           W_apply_w, W_apply_b, edge_index):
    src = edge_index[0]
    dst = edge_index[1]

    wms_t = W_msg_w[:, 0:D].T
    wme_t = W_msg_w[:, D:D + DE].T
    wmd_t = W_msg_w[:, D + DE:].T
    wes_t = W_edge_w[:, 0:D].T
    wee_t = W_edge_w[:, D:D + DE].T
    weh_t = W_edge_w[:, D + DE:].T
    wad_t = W_apply_w[:, 0:D].T
    wah_t = W_apply_w[:, D:D + DH].T
    wae_t = W_apply_w[:, D + DH:].T

    nb = 2000
    gridn = N // nb

    a_proj, b_proj, p_proj = pl.pallas_call(
        _tc1a_body,
        grid=(gridn,),
        in_specs=[_node_block(nb), _node_block(nb), _full_block(),
                  _full_block(), _full_block(), _full_block()],
        out_specs=[_node_block(nb), _node_block(nb), _node_block(nb)],
        out_shape=[jax.ShapeDtypeStruct((N, DH), jnp.float32),
                   jax.ShapeDtypeStruct((N, DH), jnp.float32),
                   jax.ShapeDtypeStruct((N, EOUT), jnp.float32)],
    )(x_src, x_dst, wms_t, wmd_t, wes_t, W_msg_b.reshape(1, DH))

    eb = 4000
    gride = E // eb
    ce_proj, re_proj = pl.pallas_call(
        _tc1b_body,
        grid=(gride,),
        in_specs=[_node_block(eb), _full_block(), _full_block(),
                  _full_block()],
        out_specs=[_node_block(eb), _node_block(eb)],
        out_shape=[jax.ShapeDtypeStruct((E, DH), jnp.float32),
                   jax.ShapeDtypeStruct((E, EOUT), jnp.float32)],
    )(e, wme_t, wee_t, W_edge_b.reshape(1, EOUT))

    part1 = _sc1(a_proj, b_proj, ce_proj, src, dst)

    q_proj, g_node, invd = pl.pallas_call(
        _tc2_body,
        grid=(gridn,),
        in_specs=[pl.BlockSpec((None, nb, None), lambda i: (0, i, 0)),
                  _node_block(nb), _full_block(), _full_block(),
                  _full_block(), _full_block()],
        out_specs=[_node_block(nb), _node_block(nb), _node_block(nb)],
        out_shape=[jax.ShapeDtypeStruct((N, EOUT), jnp.float32),
                   jax.ShapeDtypeStruct((N, NOUT), jnp.float32),
                   jax.ShapeDtypeStruct((N, EOUT), jnp.float32)],
    )(part1, x_dst, weh_t, wad_t, wah_t, W_apply_b.reshape(1, NOUT))

    eh, part2 = _sc2(p_proj, q_proj, re_proj, src, dst)

    h_out = pl.pallas_call(
        _tc3_body,
        grid=(gridn,),
        in_specs=[pl.BlockSpec((None, nb, None), lambda i: (0, i, 0)),
                  _node_block(nb), _node_block(nb), _full_block()],
        out_specs=_node_block(nb),
        out_shape=jax.ShapeDtypeStruct((N, NOUT), jnp.float32),
    )(part2, g_node, invd, wae_t)

    return (h_out, eh)


# speedup vs baseline: 2.0948x; 2.0948x over previous
import functools

import jax
import jax.numpy as jnp
from jax import lax
from jax.experimental import pallas as pl
from jax.experimental.pallas import tpu as pltpu
from jax.experimental.pallas import tpu_sc as plsc

N = 10000
E = 320000
D = 128
DE = 16
DH = 128
NOUT = 128
EOUT = 16

NC = 2
NS = 16
NW = NC * NS
EPT = E // NS
C1 = 80
NCH1 = EPT // C1
EPW = E // NW
C2 = 80
NCH2 = EPW // C2
NP = 10240
RPT = NP // NS
DH2 = DH // 2

_HIGH = jax.lax.Precision.HIGHEST


def _tc1a_body(xs_ref, xd_ref, wms_ref, wmd_ref, wes_ref, bm_ref,
               a_ref, b_ref, p_ref):
    xs = xs_ref[...]
    xd = xd_ref[...]
    a = jnp.dot(xs, wms_ref[...], precision=_HIGH) + bm_ref[...]
    b = jnp.dot(xd, wmd_ref[...], precision=_HIGH)
    a_ref[0] = a[:, 0:DH2]
    a_ref[1] = a[:, DH2:DH]
    b_ref[0] = b[:, 0:DH2]
    b_ref[1] = b[:, DH2:DH]
    p_ref[...] = jnp.dot(xs, wes_ref[...], precision=_HIGH)


def _tc1b_body(e_ref, wme_ref, wee_ref, be_ref, ce_ref, re_ref):
    ev = e_ref[...]
    ce = jnp.dot(ev, wme_ref[...], precision=_HIGH)
    ce_ref[0] = ce[:, 0:DH2]
    ce_ref[1] = ce[:, DH2:DH]
    re_ref[...] = jnp.dot(ev, wee_ref[...], precision=_HIGH) + be_ref[...]


def _tc2_body(part_ref, degp_ref, xd_ref, weh_ref, wad_ref, wah_ref, ba_ref,
              q_ref, g_ref, invd_ref):
    hs = jnp.concatenate([part_ref[0], part_ref[1]], axis=1)
    deg = 0.5 * jnp.sum(degp_ref[...], axis=0)[:, None]
    denom = jnp.maximum(deg, 1.0)
    hn = hs / denom
    mask = (deg > 0.0).astype(jnp.float32)
    q_ref[...] = jnp.dot(hn, weh_ref[...], precision=_HIGH)
    g_ref[...] = mask * (jnp.dot(xd_ref[...], wad_ref[...], precision=_HIGH)
                         + jnp.dot(hn, wah_ref[...], precision=_HIGH)
                         + ba_ref[...])
    invd_ref[...] = jnp.broadcast_to(1.0 / denom, invd_ref.shape)


def _tc3_body(sp_ref, g_ref, invd_ref, wae_ref, out_ref):
    s = sp_ref[0] + sp_ref[1]
    out_ref[...] = g_ref[...] + (jnp.dot(s, wae_ref[...], precision=_HIGH)
                                 * invd_ref[:, 0:1])


def _node_block(nb, w):
    return pl.BlockSpec((nb, w), lambda i: (i, 0))


def _full_block(shape):
    return pl.BlockSpec(shape, lambda i: tuple(0 for _ in shape))


def _sc1_body(a_hbm, b_hbm, ce_hbm, src_hbm, dst_hbm, out_hbm, deg_hbm,
              sidx1, didx, abuf, bbuf, cbuf, mbuf, degbuf, acc):
    cid = lax.axis_index("c")
    sid = lax.axis_index("s")
    wid = sid * NC + cid

    def zrow(r, _):
        for g in range(DH2 // 16):
            mbuf[r, pl.ds(g * 16, 16)] = jnp.zeros((16,), jnp.float32)
        return 0
    lax.fori_loop(0, C1, zrow, 0)
    base_r = sid * RPT
    for j in range(RPT // C1):
        pltpu.sync_copy(mbuf, acc.at[pl.ds(base_r + j * C1, C1)])

    def zdeg(i, _):
        degbuf[pl.ds(i * 16, 16)] = jnp.zeros((16,), jnp.float32)
        return 0
    lax.fori_loop(0, NP // 16, zdeg, 0)

    ebase = sid * EPT
    pltpu.sync_copy(src_hbm.at[pl.ds(ebase, EPT)], sidx1)
    plsc.subcore_barrier()

    ones = jnp.full((16,), 1.0, jnp.float32)

    def chunk_body(k, _):
        pltpu.sync_copy(dst_hbm.at[pl.ds(ebase + k * C1, C1)], didx)
        pltpu.sync_copy(a_hbm.at[cid].at[sidx1.at[pl.ds(k * C1, C1)]], abuf)
        pltpu.sync_copy(b_hbm.at[cid].at[didx], bbuf)
        pltpu.sync_copy(ce_hbm.at[cid, sid, k], cbuf)

        def row(r, _):
            for g in range(DH2 // 16):
                sl = pl.ds(g * 16, 16)
                v = abuf[r, sl] + bbuf[r, sl] + cbuf[r, sl]
                mbuf[r, sl] = jnp.maximum(v, jnp.float32(0.0))
            return 0
        lax.fori_loop(0, C1, row, 0)

        def dgrp(g, _):
            idxv = didx[pl.ds(g * 16, 16)]
            plsc.addupdate_scatter(degbuf, [idxv], ones)
            return 0
        lax.fori_loop(0, C1 // 16, dgrp, 0)

        pltpu.sync_copy(mbuf, acc.at[didx], add=True)
        return 0

    lax.fori_loop(0, NCH1, chunk_body, 0)
    plsc.subcore_barrier()

    pltpu.sync_copy(acc.at[pl.ds(base_r, RPT)],
                    out_hbm.at[cid, pl.ds(base_r, RPT)])
    pltpu.sync_copy(degbuf, deg_hbm.at[wid])


def _sc2_body(p_hbm, q_hbm, re_hbm, src_hbm, dst_hbm, eh_hbm, out_hbm,
              sidx1, didx, pbuf, qbuf, rbuf, ebuf, acc):
    cid = lax.axis_index("c")
    sid = lax.axis_index("s")
    wid = sid * NC + cid

    def zrow(r, _):
        ebuf[r, pl.ds(0, 16)] = jnp.zeros((16,), jnp.float32)
        return 0
    lax.fori_loop(0, C2, zrow, 0)
    base_r = sid * RPT
    for j in range(RPT // C2):
        pltpu.sync_copy(ebuf, acc.at[pl.ds(base_r + j * C2, C2)])

    ebase = wid * EPW
    pltpu.sync_copy(src_hbm.at[pl.ds(ebase, EPW)], sidx1)
    plsc.subcore_barrier()

    def chunk_body(k, _):
        pltpu.sync_copy(dst_hbm.at[pl.ds(ebase + k * C2, C2)], didx)
        pltpu.sync_copy(p_hbm.at[sidx1.at[pl.ds(k * C2, C2)]], pbuf)
        pltpu.sync_copy(q_hbm.at[didx], qbuf)
        pltpu.sync_copy(re_hbm.at[sid, cid, k], rbuf)

        def row(r, _):
            sl = pl.ds(0, 16)
            v = pbuf[r, sl] + qbuf[r, sl] + rbuf[r, sl]
            ebuf[r, sl] = jnp.maximum(v, jnp.float32(0.0))
            return 0
        lax.fori_loop(0, C2, row, 0)

        pltpu.sync_copy(ebuf, eh_hbm.at[sid, cid, k])
        pltpu.sync_copy(ebuf, acc.at[didx], add=True)
        return 0

    lax.fori_loop(0, NCH2, chunk_body, 0)
    plsc.subcore_barrier()

    pltpu.sync_copy(acc.at[pl.ds(base_r, RPT)],
                    out_hbm.at[cid, pl.ds(base_r, RPT)])


_sc_mesh = plsc.VectorSubcoreMesh(core_axis_name="c", subcore_axis_name="s",
                                  num_cores=NC, num_subcores=NS)

_sc1 = functools.partial(
    pl.kernel, _sc1_body,
    out_type=[jax.ShapeDtypeStruct((NC, NP, DH2), jnp.float32),
              jax.ShapeDtypeStruct((NW, NP), jnp.float32)],
    mesh=_sc_mesh,
    compiler_params=pltpu.CompilerParams(use_tc_tiling_on_sc=False,
                                         needs_layout_passes=False),
    scratch_types=[
        pltpu.VMEM((EPT,), jnp.int32),
        pltpu.VMEM((C1,), jnp.int32),
        pltpu.VMEM((C1, DH2), jnp.float32),
        pltpu.VMEM((C1, DH2), jnp.float32),
        pltpu.VMEM((C1, DH2), jnp.float32),
        pltpu.VMEM((C1, DH2), jnp.float32),
        pltpu.VMEM((NP,), jnp.float32),
        pltpu.VMEM_SHARED((NP, DH2), jnp.float32),
    ],
)()

_sc2 = functools.partial(
    pl.kernel, _sc2_body,
    out_type=[jax.ShapeDtypeStruct((NS, NC, NCH2, C2, EOUT), jnp.float32),
              jax.ShapeDtypeStruct((NC, NP, EOUT), jnp.float32)],
    mesh=_sc_mesh,
    compiler_params=pltpu.CompilerParams(use_tc_tiling_on_sc=False,
                                         needs_layout_passes=False),
    scratch_types=[
        pltpu.VMEM((EPW,), jnp.int32),
        pltpu.VMEM((C2,), jnp.int32),
        pltpu.VMEM((C2, EOUT), jnp.float32),
        pltpu.VMEM((C2, EOUT), jnp.float32),
        pltpu.VMEM((C2, EOUT), jnp.float32),
        pltpu.VMEM((C2, EOUT), jnp.float32),
        pltpu.VMEM_SHARED((NP, EOUT), jnp.float32),
    ],
)()


def kernel(x_src, x_dst, e, W_msg_w, W_msg_b, W_edge_w, W_edge_b,
           W_apply_w, W_apply_b, edge_index):
    src = edge_index[0]
    dst = edge_index[1]

    wms_t = W_msg_w[:, 0:D].T
    wme_t = W_msg_w[:, D:D + DE].T
    wmd_t = W_msg_w[:, D + DE:].T
    wes_t = W_edge_w[:, 0:D].T
    wee_t = W_edge_w[:, D:D + DE].T
    weh_t = W_edge_w[:, D + DE:].T
    wad_t = W_apply_w[:, 0:D].T
    wah_t = W_apply_w[:, D:D + DH].T
    wae_t = W_apply_w[:, D + DH:].T

    nb = 2048
    gridn = pl.cdiv(N, nb)

    a_proj, b_proj, p_proj = pl.pallas_call(
        _tc1a_body,
        grid=(gridn,),
        in_specs=[_node_block(nb, D), _node_block(nb, D),
                  _full_block((D, DH)), _full_block((D, DH)),
                  _full_block((D, EOUT)), _full_block((1, DH))],
        out_specs=[pl.BlockSpec((NC, nb, DH2), lambda i: (0, i, 0)),
                   pl.BlockSpec((NC, nb, DH2), lambda i: (0, i, 0)),
                   _node_block(nb, EOUT)],
        out_shape=[jax.ShapeDtypeStruct((NC, N, DH2), jnp.float32),
                   jax.ShapeDtypeStruct((NC, N, DH2), jnp.float32),
                   jax.ShapeDtypeStruct((N, EOUT), jnp.float32)],
    )(x_src, x_dst, wms_t, wmd_t, wes_t, W_msg_b.reshape(1, DH))

    eb = 4000
    gride = E // eb
    ce_proj, re_proj = pl.pallas_call(
        _tc1b_body,
        grid=(gride,),
        in_specs=[_node_block(eb, DE), _full_block((DE, DH)),
                  _full_block((DE, EOUT)), _full_block((1, EOUT))],
        out_specs=[pl.BlockSpec((NC, eb, DH2), lambda i: (0, i, 0)),
                   _node_block(eb, EOUT)],
        out_shape=[jax.ShapeDtypeStruct((NC, E, DH2), jnp.float32),
                   jax.ShapeDtypeStruct((E, EOUT), jnp.float32)],
    )(e, wme_t, wee_t, W_edge_b.reshape(1, EOUT))

    ce1 = ce_proj.reshape(NC, NS, NCH1, C1, DH2)

    part1, deg_parts = _sc1(a_proj, b_proj, ce1, src, dst)

    q_proj, g_node, invd = pl.pallas_call(
        _tc2_body,
        grid=(gridn,),
        in_specs=[pl.BlockSpec((NC, nb, DH2), lambda i: (0, i, 0)),
                  pl.BlockSpec((NW, nb), lambda i: (0, i)),
                  _node_block(nb, D), _full_block((DH, EOUT)),
                  _full_block((D, NOUT)), _full_block((DH, NOUT)),
                  _full_block((1, NOUT))],
        out_specs=[_node_block(nb, EOUT), _node_block(nb, NOUT),
                   _node_block(nb, EOUT)],
        out_shape=[jax.ShapeDtypeStruct((N, EOUT), jnp.float32),
                   jax.ShapeDtypeStruct((N, NOUT), jnp.float32),
                   jax.ShapeDtypeStruct((N, EOUT), jnp.float32)],
    )(part1, deg_parts, x_dst, weh_t, wad_t, wah_t,
      W_apply_b.reshape(1, NOUT))

    re2 = re_proj.reshape(NS, NC, NCH2, C2, EOUT)

    eh5, part2 = _sc2(p_proj, q_proj, re2, src, dst)
    eh = eh5.reshape(E, EOUT)

    h_out = pl.pallas_call(
        _tc3_body,
        grid=(gridn,),
        in_specs=[pl.BlockSpec((NC, nb, EOUT), lambda i: (0, i, 0)),
                  _node_block(nb, NOUT), _node_block(nb, EOUT),
                  _full_block((EOUT, NOUT))],
        out_specs=_node_block(nb, NOUT),
        out_shape=jax.ShapeDtypeStruct((N, NOUT), jnp.float32),
    )(part2, g_node, invd, wae_t)

    return (h_out, eh)


# speedup vs baseline: 3.3734x; 1.6103x over previous
import functools

import jax
import jax.numpy as jnp
from jax import lax
from jax.experimental import pallas as pl
from jax.experimental.pallas import tpu as pltpu
from jax.experimental.pallas import tpu_sc as plsc

N = 10000
E = 320000
D = 128
DE = 16
DH = 128
NOUT = 128
EOUT = 16

NC = 2
NS = 16
NW = NC * NS
EPT = E // NS
C1 = 80
NCH1 = EPT // C1
EPW = E // NW
C2 = 80
NCH2 = EPW // C2
NP = 10240
RPT = NP // NS
DH2 = DH // 2

_HIGH = jax.lax.Precision.HIGHEST


def _tc1a_body(xs_ref, xd_ref, wms_ref, wmd_ref, wes_ref, bm_ref,
               a_ref, b_ref, p_ref):
    xs = xs_ref[...]
    xd = xd_ref[...]
    a = jnp.dot(xs, wms_ref[...], precision=_HIGH) + bm_ref[...]
    b = jnp.dot(xd, wmd_ref[...], precision=_HIGH)
    a_ref[0] = a[:, 0:DH2]
    a_ref[1] = a[:, DH2:DH]
    b_ref[0] = b[:, 0:DH2]
    b_ref[1] = b[:, DH2:DH]
    p_ref[...] = jnp.dot(xs, wes_ref[...], precision=_HIGH)


def _tc1b_body(e_ref, wme_ref, wee_ref, be_ref, ce_ref, re_ref):
    ev = e_ref[...]
    ce = jnp.dot(ev, wme_ref[...], precision=_HIGH)
    ce_ref[0] = ce[:, 0:DH2]
    ce_ref[1] = ce[:, DH2:DH]
    re_ref[...] = jnp.dot(ev, wee_ref[...], precision=_HIGH) + be_ref[...]


def _tc2_body(part_ref, degp_ref, xd_ref, weh_ref, wad_ref, wah_ref, ba_ref,
              q_ref, g_ref, invd_ref):
    hs = jnp.concatenate([part_ref[0], part_ref[1]], axis=1)
    deg = 0.5 * jnp.sum(degp_ref[...], axis=0)[:, None]
    denom = jnp.maximum(deg, 1.0)
    hn = hs / denom
    mask = (deg > 0.0).astype(jnp.float32)
    q_ref[...] = jnp.dot(hn, weh_ref[...], precision=_HIGH)
    g_ref[...] = mask * (jnp.dot(xd_ref[...], wad_ref[...], precision=_HIGH)
                         + jnp.dot(hn, wah_ref[...], precision=_HIGH)
                         + ba_ref[...])
    invd_ref[...] = jnp.broadcast_to(1.0 / denom, invd_ref.shape)


def _tc3_body(sp_ref, g_ref, invd_ref, wae_ref, out_ref):
    s = sp_ref[0] + sp_ref[1]
    out_ref[...] = g_ref[...] + (jnp.dot(s, wae_ref[...], precision=_HIGH)
                                 * invd_ref[:, 0:1])


def _node_block(nb, w):
    return pl.BlockSpec((nb, w), lambda i: (i, 0))


def _full_block(shape):
    return pl.BlockSpec(shape, lambda i: tuple(0 for _ in shape))


def _sc1_body(a_hbm, b_hbm, ce_hbm, src_hbm, dst_hbm, out_hbm, deg_hbm,
              sidx1, didx2, abuf, bbuf, cbuf, mbuf, degbuf, acc,
              sg0, sg1, si0, si1):
    cid = lax.axis_index("c")
    sid = lax.axis_index("s")
    wid = sid * NC + cid
    sgs = (sg0, sg1)
    sis = (si0, si1)

    ebase = sid * EPT

    def gathers(k, p):
        pltpu.async_copy(a_hbm.at[cid].at[sidx1.at[pl.ds(k * C1, C1)]],
                         abuf.at[p], sgs[p])
        pltpu.async_copy(b_hbm.at[cid].at[didx2.at[p]], bbuf.at[p], sgs[p])
        pltpu.async_copy(ce_hbm.at[cid, sid, k], cbuf.at[p], sgs[p])

    pltpu.sync_copy(src_hbm.at[pl.ds(ebase, EPT)], sidx1)
    pltpu.sync_copy(dst_hbm.at[pl.ds(ebase, C1)], didx2.at[0])
    gathers(0, 0)
    pltpu.async_copy(dst_hbm.at[pl.ds(ebase + C1, C1)], didx2.at[1], si1)

    def zrow(r, _):
        for g in range(DH2 // 16):
            mbuf[r, pl.ds(g * 16, 16)] = jnp.zeros((16,), jnp.float32)
        return 0
    lax.fori_loop(0, C1, zrow, 0)
    base_r = sid * RPT
    for j in range(RPT // C1):
        pltpu.sync_copy(mbuf, acc.at[pl.ds(base_r + j * C1, C1)])

    def zdeg(i, _):
        degbuf[pl.ds(i * 16, 16)] = jnp.zeros((16,), jnp.float32)
        return 0
    lax.fori_loop(0, NP // 16, zdeg, 0)

    plsc.subcore_barrier()

    ones = jnp.full((16,), 1.0, jnp.float32)

    def chunk_body(k, p):
        pn = (p + 1) % 2
        pltpu.make_async_copy(a_hbm.at[cid].at[sidx1.at[pl.ds(k * C1, C1)]],
                              abuf.at[p], sgs[p]).wait()
        pltpu.make_async_copy(b_hbm.at[cid].at[didx2.at[p]], bbuf.at[p],
                              sgs[p]).wait()
        pltpu.make_async_copy(ce_hbm.at[cid, sid, k], cbuf.at[p],
                              sgs[p]).wait()

        @pl.when(k + 1 < NCH1)
        def _():
            pltpu.make_async_copy(dst_hbm.at[pl.ds(ebase, C1)],
                                  didx2.at[pn], sis[pn]).wait()
            gathers(k + 1, pn)

        def row(r, _):
            for g in range(DH2 // 16):
                sl = pl.ds(g * 16, 16)
                v = abuf[p, r, sl] + bbuf[p, r, sl] + cbuf[p, r, sl]
                mbuf[r, sl] = jnp.maximum(v, jnp.float32(0.0))
            return 0
        lax.fori_loop(0, C1, row, 0)

        def dgrp(g, _):
            idxv = didx2[p, pl.ds(g * 16, 16)]
            plsc.addupdate_scatter(degbuf, [idxv], ones)
            return 0
        lax.fori_loop(0, C1 // 16, dgrp, 0)

        pltpu.sync_copy(mbuf, acc.at[didx2.at[p]], add=True)

        @pl.when(k + 2 < NCH1)
        def _():
            pltpu.async_copy(dst_hbm.at[pl.ds(ebase + (k + 2) * C1, C1)],
                             didx2.at[p], sis[p])

    def step(k2, _):
        chunk_body(2 * k2, 0)
        chunk_body(2 * k2 + 1, 1)
        return 0

    lax.fori_loop(0, NCH1 // 2, step, 0)
    plsc.subcore_barrier()

    pltpu.sync_copy(acc.at[pl.ds(base_r, RPT)],
                    out_hbm.at[cid, pl.ds(base_r, RPT)])
    pltpu.sync_copy(degbuf, deg_hbm.at[wid])


def _sc2_body(p_hbm, q_hbm, re_hbm, src_hbm, dst_hbm, eh_hbm, out_hbm,
              sidx1, didx2, pbuf, qbuf, rbuf, ebuf, acc,
              sg0, sg1, si0, si1):
    cid = lax.axis_index("c")
    sid = lax.axis_index("s")
    wid = sid * NC + cid
    sgs = (sg0, sg1)
    sis = (si0, si1)

    ebase = wid * EPW

    def gathers(k, p):
        pltpu.async_copy(p_hbm.at[sidx1.at[pl.ds(k * C2, C2)]],
                         pbuf.at[p], sgs[p])
        pltpu.async_copy(q_hbm.at[didx2.at[p]], qbuf.at[p], sgs[p])
        pltpu.async_copy(re_hbm.at[sid, cid, k], rbuf.at[p], sgs[p])

    pltpu.sync_copy(src_hbm.at[pl.ds(ebase, EPW)], sidx1)
    pltpu.sync_copy(dst_hbm.at[pl.ds(ebase, C2)], didx2.at[0])
    gathers(0, 0)
    pltpu.async_copy(dst_hbm.at[pl.ds(ebase + C2, C2)], didx2.at[1], si1)

    def zrow(r, _):
        ebuf[r, pl.ds(0, 16)] = jnp.zeros((16,), jnp.float32)
        return 0
    lax.fori_loop(0, C2, zrow, 0)
    base_r = sid * RPT
    for j in range(RPT // C2):
        pltpu.sync_copy(ebuf, acc.at[pl.ds(base_r + j * C2, C2)])

    plsc.subcore_barrier()

    def chunk_body(k, p):
        pn = (p + 1) % 2
        pltpu.make_async_copy(p_hbm.at[sidx1.at[pl.ds(k * C2, C2)]],
                              pbuf.at[p], sgs[p]).wait()
        pltpu.make_async_copy(q_hbm.at[didx2.at[p]], qbuf.at[p],
                              sgs[p]).wait()
        pltpu.make_async_copy(re_hbm.at[sid, cid, k], rbuf.at[p],
                              sgs[p]).wait()

        @pl.when(k + 1 < NCH2)
        def _():
            pltpu.make_async_copy(dst_hbm.at[pl.ds(ebase, C2)],
                                  didx2.at[pn], sis[pn]).wait()
            gathers(k + 1, pn)

        def row(r, _):
            sl = pl.ds(0, 16)
            v = pbuf[p, r, sl] + qbuf[p, r, sl] + rbuf[p, r, sl]
            ebuf[r, sl] = jnp.maximum(v, jnp.float32(0.0))
            return 0
        lax.fori_loop(0, C2, row, 0)

        pltpu.sync_copy(ebuf, eh_hbm.at[sid, cid, k])
        pltpu.sync_copy(ebuf, acc.at[didx2.at[p]], add=True)

        @pl.when(k + 2 < NCH2)
        def _():
            pltpu.async_copy(dst_hbm.at[pl.ds(ebase + (k + 2) * C2, C2)],
                             didx2.at[p], sis[p])

    def step(k2, _):
        chunk_body(2 * k2, 0)
        chunk_body(2 * k2 + 1, 1)
        return 0

    lax.fori_loop(0, NCH2 // 2, step, 0)
    chunk_body(NCH2 - 1, 0)
    plsc.subcore_barrier()

    pltpu.sync_copy(acc.at[pl.ds(base_r, RPT)],
                    out_hbm.at[cid, pl.ds(base_r, RPT)])


_sc_mesh = plsc.VectorSubcoreMesh(core_axis_name="c", subcore_axis_name="s",
                                  num_cores=NC, num_subcores=NS)

_sc1 = functools.partial(
    pl.kernel, _sc1_body,
    out_type=[jax.ShapeDtypeStruct((NC, NP, DH2), jnp.float32),
              jax.ShapeDtypeStruct((NW, NP), jnp.float32)],
    mesh=_sc_mesh,
    compiler_params=pltpu.CompilerParams(use_tc_tiling_on_sc=False,
                                         needs_layout_passes=False),
    scratch_types=[
        pltpu.VMEM((EPT,), jnp.int32),
        pltpu.VMEM((2, C1), jnp.int32),
        pltpu.VMEM((2, C1, DH2), jnp.float32),
        pltpu.VMEM((2, C1, DH2), jnp.float32),
        pltpu.VMEM((2, C1, DH2), jnp.float32),
        pltpu.VMEM((C1, DH2), jnp.float32),
        pltpu.VMEM((NP,), jnp.float32),
        pltpu.VMEM_SHARED((NP, DH2), jnp.float32),
        pltpu.SemaphoreType.DMA,
        pltpu.SemaphoreType.DMA,
        pltpu.SemaphoreType.DMA,
        pltpu.SemaphoreType.DMA,
    ],
)()

_sc2 = functools.partial(
    pl.kernel, _sc2_body,
    out_type=[jax.ShapeDtypeStruct((NS, NC, NCH2, C2, EOUT), jnp.float32),
              jax.ShapeDtypeStruct((NC, NP, EOUT), jnp.float32)],
    mesh=_sc_mesh,
    compiler_params=pltpu.CompilerParams(use_tc_tiling_on_sc=False,
                                         needs_layout_passes=False),
    scratch_types=[
        pltpu.VMEM((EPW,), jnp.int32),
        pltpu.VMEM((2, C2), jnp.int32),
        pltpu.VMEM((2, C2, EOUT), jnp.float32),
        pltpu.VMEM((2, C2, EOUT), jnp.float32),
        pltpu.VMEM((2, C2, EOUT), jnp.float32),
        pltpu.VMEM((C2, EOUT), jnp.float32),
        pltpu.VMEM_SHARED((NP, EOUT), jnp.float32),
        pltpu.SemaphoreType.DMA,
        pltpu.SemaphoreType.DMA,
        pltpu.SemaphoreType.DMA,
        pltpu.SemaphoreType.DMA,
    ],
)()


def kernel(x_src, x_dst, e, W_msg_w, W_msg_b, W_edge_w, W_edge_b,
           W_apply_w, W_apply_b, edge_index):
    src = edge_index[0]
    dst = edge_index[1]

    wms_t = W_msg_w[:, 0:D].T
    wme_t = W_msg_w[:, D:D + DE].T
    wmd_t = W_msg_w[:, D + DE:].T
    wes_t = W_edge_w[:, 0:D].T
    wee_t = W_edge_w[:, D:D + DE].T
    weh_t = W_edge_w[:, D + DE:].T
    wad_t = W_apply_w[:, 0:D].T
    wah_t = W_apply_w[:, D:D + DH].T
    wae_t = W_apply_w[:, D + DH:].T

    nb = 2048
    gridn = pl.cdiv(N, nb)

    a_proj, b_proj, p_proj = pl.pallas_call(
        _tc1a_body,
        grid=(gridn,),
        in_specs=[_node_block(nb, D), _node_block(nb, D),
                  _full_block((D, DH)), _full_block((D, DH)),
                  _full_block((D, EOUT)), _full_block((1, DH))],
        out_specs=[pl.BlockSpec((NC, nb, DH2), lambda i: (0, i, 0)),
                   pl.BlockSpec((NC, nb, DH2), lambda i: (0, i, 0)),
                   _node_block(nb, EOUT)],
        out_shape=[jax.ShapeDtypeStruct((NC, N, DH2), jnp.float32),
                   jax.ShapeDtypeStruct((NC, N, DH2), jnp.float32),
                   jax.ShapeDtypeStruct((N, EOUT), jnp.float32)],
    )(x_src, x_dst, wms_t, wmd_t, wes_t, W_msg_b.reshape(1, DH))

    eb = 4000
    gride = E // eb
    ce_proj, re_proj = pl.pallas_call(
        _tc1b_body,
        grid=(gride,),
        in_specs=[_node_block(eb, DE), _full_block((DE, DH)),
                  _full_block((DE, EOUT)), _full_block((1, EOUT))],
        out_specs=[pl.BlockSpec((NC, eb, DH2), lambda i: (0, i, 0)),
                   _node_block(eb, EOUT)],
        out_shape=[jax.ShapeDtypeStruct((NC, E, DH2), jnp.float32),
                   jax.ShapeDtypeStruct((E, EOUT), jnp.float32)],
    )(e, wme_t, wee_t, W_edge_b.reshape(1, EOUT))

    ce1 = ce_proj.reshape(NC, NS, NCH1, C1, DH2)

    part1, deg_parts = _sc1(a_proj, b_proj, ce1, src, dst)

    q_proj, g_node, invd = pl.pallas_call(
        _tc2_body,
        grid=(gridn,),
        in_specs=[pl.BlockSpec((NC, nb, DH2), lambda i: (0, i, 0)),
                  pl.BlockSpec((NW, nb), lambda i: (0, i)),
                  _node_block(nb, D), _full_block((DH, EOUT)),
                  _full_block((D, NOUT)), _full_block((DH, NOUT)),
                  _full_block((1, NOUT))],
        out_specs=[_node_block(nb, EOUT), _node_block(nb, NOUT),
                   _node_block(nb, EOUT)],
        out_shape=[jax.ShapeDtypeStruct((N, EOUT), jnp.float32),
                   jax.ShapeDtypeStruct((N, NOUT), jnp.float32),
                   jax.ShapeDtypeStruct((N, EOUT), jnp.float32)],
    )(part1, deg_parts, x_dst, weh_t, wad_t, wah_t,
      W_apply_b.reshape(1, NOUT))

    re2 = re_proj.reshape(NS, NC, NCH2, C2, EOUT)

    eh5, part2 = _sc2(p_proj, q_proj, re2, src, dst)
    eh = eh5.reshape(E, EOUT)

    h_out = pl.pallas_call(
        _tc3_body,
        grid=(gridn,),
        in_specs=[pl.BlockSpec((NC, nb, EOUT), lambda i: (0, i, 0)),
                  _node_block(nb, NOUT), _node_block(nb, EOUT),
                  _full_block((EOUT, NOUT))],
        out_specs=_node_block(nb, NOUT),
        out_shape=jax.ShapeDtypeStruct((N, NOUT), jnp.float32),
    )(part2, g_node, invd, wae_t)

    return (h_out, eh)


# speedup vs baseline: 3.3807x; 1.0022x over previous
import functools

import jax
import jax.numpy as jnp
from jax import lax
from jax.experimental import pallas as pl
from jax.experimental.pallas import tpu as pltpu
from jax.experimental.pallas import tpu_sc as plsc

N = 10000
E = 320000
D = 128
DE = 16
DH = 128
NOUT = 128
EOUT = 16

NC = 2
NS = 16
NW = NC * NS
EPT = E // NS
C1 = 80
NCH1 = EPT // C1
EPW = E // NW
C2 = 80
NCH2 = EPW // C2
NP = 10240
RPT = NP // NS
DH2 = DH // 2

_HIGH = jax.lax.Precision.HIGHEST


def _tc1a_body(xs_ref, xd_ref, wms_ref, wmd_ref, wes_ref, bm_ref,
               a_ref, b_ref, p_ref):
    xs = xs_ref[...]
    xd = xd_ref[...]
    a = jnp.dot(xs, wms_ref[...], precision=_HIGH) + bm_ref[...]
    b = jnp.dot(xd, wmd_ref[...], precision=_HIGH)
    a_ref[0] = a[:, 0:DH2]
    a_ref[1] = a[:, DH2:DH]
    b_ref[0] = b[:, 0:DH2]
    b_ref[1] = b[:, DH2:DH]
    p_ref[...] = jnp.dot(xs, wes_ref[...], precision=_HIGH)


def _tc1b_body(e_ref, wme_ref, wee_ref, be_ref, ce_ref, re_ref):
    ev = e_ref[...]
    ce = jnp.dot(ev, wme_ref[...], precision=_HIGH)
    ce_ref[0] = ce[:, 0:DH2]
    ce_ref[1] = ce[:, DH2:DH]
    re_ref[...] = jnp.dot(ev, wee_ref[...], precision=_HIGH) + be_ref[...]


def _tc2_body(part_ref, degp_ref, xd_ref, weh_ref, wad_ref, wah_ref, ba_ref,
              q_ref, g_ref, invd_ref):
    hs = jnp.concatenate([part_ref[0], part_ref[1]], axis=1)
    deg = 0.5 * jnp.sum(degp_ref[...], axis=0)[:, None]
    denom = jnp.maximum(deg, 1.0)
    hn = hs / denom
    mask = (deg > 0.0).astype(jnp.float32)
    q_ref[...] = jnp.dot(hn, weh_ref[...], precision=_HIGH)
    g_ref[...] = mask * (jnp.dot(xd_ref[...], wad_ref[...], precision=_HIGH)
                         + jnp.dot(hn, wah_ref[...], precision=_HIGH)
                         + ba_ref[...])
    invd_ref[...] = jnp.broadcast_to(1.0 / denom, invd_ref.shape)


def _tc3_body(sp_ref, g_ref, invd_ref, wae_ref, out_ref):
    s = sp_ref[0] + sp_ref[1]
    out_ref[...] = g_ref[...] + (jnp.dot(s, wae_ref[...], precision=_HIGH)
                                 * invd_ref[:, 0:1])


def _node_block(nb, w):
    return pl.BlockSpec((nb, w), lambda i: (i, 0))


def _full_block(shape):
    return pl.BlockSpec(shape, lambda i: tuple(0 for _ in shape))


def _sc1_body(a_hbm, b_hbm, ce_hbm, src_hbm, dst_hbm, out_hbm, deg_hbm,
              sidx1, didx2, abuf, bbuf, cbuf, mbuf, degbuf, acc,
              sg0, sg1, si0, si1):
    cid = lax.axis_index("c")
    sid = lax.axis_index("s")
    wid = sid * NC + cid
    sgs = (sg0, sg1)
    sis = (si0, si1)

    ebase = sid * EPT

    def gathers(k, p):
        pltpu.async_copy(a_hbm.at[cid].at[sidx1.at[pl.ds(k * C1, C1)]],
                         abuf.at[p], sgs[p])
        pltpu.async_copy(b_hbm.at[cid].at[didx2.at[p]], bbuf.at[p], sgs[p])
        pltpu.async_copy(ce_hbm.at[cid].at[pl.ds(ebase + k * C1, C1)],
                         cbuf.at[p], sgs[p])

    pltpu.sync_copy(src_hbm.at[pl.ds(ebase, EPT)], sidx1)
    pltpu.sync_copy(dst_hbm.at[pl.ds(ebase, C1)], didx2.at[0])
    gathers(0, 0)
    pltpu.async_copy(dst_hbm.at[pl.ds(ebase + C1, C1)], didx2.at[1], si1)

    def zrow(r, _):
        for g in range(DH2 // 16):
            mbuf[r, pl.ds(g * 16, 16)] = jnp.zeros((16,), jnp.float32)
        return 0
    lax.fori_loop(0, C1, zrow, 0)
    base_r = sid * RPT
    for j in range(RPT // C1):
        pltpu.sync_copy(mbuf, acc.at[pl.ds(base_r + j * C1, C1)])

    def zdeg(i, _):
        degbuf[pl.ds(i * 16, 16)] = jnp.zeros((16,), jnp.float32)
        return 0
    lax.fori_loop(0, NP // 16, zdeg, 0)

    plsc.subcore_barrier()

    ones = jnp.full((16,), 1.0, jnp.float32)

    def chunk_body(k, p):
        pn = (p + 1) % 2
        pltpu.make_async_copy(a_hbm.at[cid].at[sidx1.at[pl.ds(k * C1, C1)]],
                              abuf.at[p], sgs[p]).wait()
        pltpu.make_async_copy(b_hbm.at[cid].at[didx2.at[p]], bbuf.at[p],
                              sgs[p]).wait()
        pltpu.make_async_copy(ce_hbm.at[cid].at[pl.ds(ebase + k * C1, C1)],
                              cbuf.at[p], sgs[p]).wait()

        @pl.when(k + 1 < NCH1)
        def _():
            pltpu.make_async_copy(dst_hbm.at[pl.ds(ebase, C1)],
                                  didx2.at[pn], sis[pn]).wait()
            gathers(k + 1, pn)

        def row(r, _):
            for g in range(DH2 // 16):
                sl = pl.ds(g * 16, 16)
                v = abuf[p, r, sl] + bbuf[p, r, sl] + cbuf[p, r, sl]
                mbuf[r, sl] = jnp.maximum(v, jnp.float32(0.0))
            return 0
        lax.fori_loop(0, C1, row, 0)

        def dgrp(g, _):
            idxv = didx2[p, pl.ds(g * 16, 16)]
            plsc.addupdate_scatter(degbuf, [idxv], ones)
            return 0
        lax.fori_loop(0, C1 // 16, dgrp, 0)

        pltpu.sync_copy(mbuf, acc.at[didx2.at[p]], add=True)

        @pl.when(k + 2 < NCH1)
        def _():
            pltpu.async_copy(dst_hbm.at[pl.ds(ebase + (k + 2) * C1, C1)],
                             didx2.at[p], sis[p])

    def step(k2, _):
        chunk_body(2 * k2, 0)
        chunk_body(2 * k2 + 1, 1)
        return 0

    lax.fori_loop(0, NCH1 // 2, step, 0)
    plsc.subcore_barrier()

    pltpu.sync_copy(acc.at[pl.ds(base_r, RPT)],
                    out_hbm.at[cid, pl.ds(base_r, RPT)])
    pltpu.sync_copy(degbuf, deg_hbm.at[wid])


def _sc2_body(p_hbm, q_hbm, re_hbm, src_hbm, dst_hbm, eh_hbm, out_hbm,
              sidx1, didx2, pbuf, qbuf, rbuf, ebuf, acc,
              sg0, sg1, si0, si1):
    cid = lax.axis_index("c")
    sid = lax.axis_index("s")
    wid = sid * NC + cid
    sgs = (sg0, sg1)
    sis = (si0, si1)

    ebase = wid * EPW

    def gathers(k, p):
        pltpu.async_copy(p_hbm.at[sidx1.at[pl.ds(k * C2, C2)]],
                         pbuf.at[p], sgs[p])
        pltpu.async_copy(q_hbm.at[didx2.at[p]], qbuf.at[p], sgs[p])
        pltpu.async_copy(re_hbm.at[pl.ds(ebase + k * C2, C2)],
                         rbuf.at[p], sgs[p])

    pltpu.sync_copy(src_hbm.at[pl.ds(ebase, EPW)], sidx1)
    pltpu.sync_copy(dst_hbm.at[pl.ds(ebase, C2)], didx2.at[0])
    gathers(0, 0)
    pltpu.async_copy(dst_hbm.at[pl.ds(ebase + C2, C2)], didx2.at[1], si1)

    def zrow(r, _):
        ebuf[r, pl.ds(0, 16)] = jnp.zeros((16,), jnp.float32)
        return 0
    lax.fori_loop(0, C2, zrow, 0)
    base_r = sid * RPT
    for j in range(RPT // C2):
        pltpu.sync_copy(ebuf, acc.at[pl.ds(base_r + j * C2, C2)])

    plsc.subcore_barrier()

    def chunk_body(k, p):
        pn = (p + 1) % 2
        pltpu.make_async_copy(p_hbm.at[sidx1.at[pl.ds(k * C2, C2)]],
                              pbuf.at[p], sgs[p]).wait()
        pltpu.make_async_copy(q_hbm.at[didx2.at[p]], qbuf.at[p],
                              sgs[p]).wait()
        pltpu.make_async_copy(re_hbm.at[pl.ds(ebase + k * C2, C2)],
                              rbuf.at[p], sgs[p]).wait()

        @pl.when(k + 1 < NCH2)
        def _():
            pltpu.make_async_copy(dst_hbm.at[pl.ds(ebase, C2)],
                                  didx2.at[pn], sis[pn]).wait()
            gathers(k + 1, pn)

        def row(r, _):
            sl = pl.ds(0, 16)
            v = pbuf[p, r, sl] + qbuf[p, r, sl] + rbuf[p, r, sl]
            ebuf[r, sl] = jnp.maximum(v, jnp.float32(0.0))
            return 0
        lax.fori_loop(0, C2, row, 0)

        pltpu.sync_copy(ebuf, eh_hbm.at[pl.ds(ebase + k * C2, C2)])
        pltpu.sync_copy(ebuf, acc.at[didx2.at[p]], add=True)

        @pl.when(k + 2 < NCH2)
        def _():
            pltpu.async_copy(dst_hbm.at[pl.ds(ebase + (k + 2) * C2, C2)],
                             didx2.at[p], sis[p])

    def step(k2, _):
        chunk_body(2 * k2, 0)
        chunk_body(2 * k2 + 1, 1)
        return 0

    lax.fori_loop(0, NCH2 // 2, step, 0)
    chunk_body(NCH2 - 1, 0)
    plsc.subcore_barrier()

    pltpu.sync_copy(acc.at[pl.ds(base_r, RPT)],
                    out_hbm.at[cid, pl.ds(base_r, RPT)])


_sc_mesh = plsc.VectorSubcoreMesh(core_axis_name="c", subcore_axis_name="s",
                                  num_cores=NC, num_subcores=NS)

_sc1 = functools.partial(
    pl.kernel, _sc1_body,
    out_type=[jax.ShapeDtypeStruct((NC, NP, DH2), jnp.float32),
              jax.ShapeDtypeStruct((NW, NP), jnp.float32)],
    mesh=_sc_mesh,
    compiler_params=pltpu.CompilerParams(use_tc_tiling_on_sc=False,
                                         needs_layout_passes=False),
    scratch_types=[
        pltpu.VMEM((EPT,), jnp.int32),
        pltpu.VMEM((2, C1), jnp.int32),
        pltpu.VMEM((2, C1, DH2), jnp.float32),
        pltpu.VMEM((2, C1, DH2), jnp.float32),
        pltpu.VMEM((2, C1, DH2), jnp.float32),
        pltpu.VMEM((C1, DH2), jnp.float32),
        pltpu.VMEM((NP,), jnp.float32),
        pltpu.VMEM_SHARED((NP, DH2), jnp.float32),
        pltpu.SemaphoreType.DMA,
        pltpu.SemaphoreType.DMA,
        pltpu.SemaphoreType.DMA,
        pltpu.SemaphoreType.DMA,
    ],
)()

_sc2 = functools.partial(
    pl.kernel, _sc2_body,
    out_type=[jax.ShapeDtypeStruct((E, EOUT), jnp.float32),
              jax.ShapeDtypeStruct((NC, NP, EOUT), jnp.float32)],
    mesh=_sc_mesh,
    compiler_params=pltpu.CompilerParams(use_tc_tiling_on_sc=False,
                                         needs_layout_passes=False),
    scratch_types=[
        pltpu.VMEM((EPW,), jnp.int32),
        pltpu.VMEM((2, C2), jnp.int32),
        pltpu.VMEM((2, C2, EOUT), jnp.float32),
        pltpu.VMEM((2, C2, EOUT), jnp.float32),
        pltpu.VMEM((2, C2, EOUT), jnp.float32),
        pltpu.VMEM((C2, EOUT), jnp.float32),
        pltpu.VMEM_SHARED((NP, EOUT), jnp.float32),
        pltpu.SemaphoreType.DMA,
        pltpu.SemaphoreType.DMA,
        pltpu.SemaphoreType.DMA,
        pltpu.SemaphoreType.DMA,
    ],
)()


def kernel(x_src, x_dst, e, W_msg_w, W_msg_b, W_edge_w, W_edge_b,
           W_apply_w, W_apply_b, edge_index):
    src = edge_index[0]
    dst = edge_index[1]

    wms_t = W_msg_w[:, 0:D].T
    wme_t = W_msg_w[:, D:D + DE].T
    wmd_t = W_msg_w[:, D + DE:].T
    wes_t = W_edge_w[:, 0:D].T
    wee_t = W_edge_w[:, D:D + DE].T
    weh_t = W_edge_w[:, D + DE:].T
    wad_t = W_apply_w[:, 0:D].T
    wah_t = W_apply_w[:, D:D + DH].T
    wae_t = W_apply_w[:, D + DH:].T

    nb = 2048
    gridn = pl.cdiv(N, nb)

    a_proj, b_proj, p_proj = pl.pallas_call(
        _tc1a_body,
        grid=(gridn,),
        in_specs=[_node_block(nb, D), _node_block(nb, D),
                  _full_block((D, DH)), _full_block((D, DH)),
                  _full_block((D, EOUT)), _full_block((1, DH))],
        out_specs=[pl.BlockSpec((NC, nb, DH2), lambda i: (0, i, 0)),
                   pl.BlockSpec((NC, nb, DH2), lambda i: (0, i, 0)),
                   _node_block(nb, EOUT)],
        out_shape=[jax.ShapeDtypeStruct((NC, N, DH2), jnp.float32),
                   jax.ShapeDtypeStruct((NC, N, DH2), jnp.float32),
                   jax.ShapeDtypeStruct((N, EOUT), jnp.float32)],
    )(x_src, x_dst, wms_t, wmd_t, wes_t, W_msg_b.reshape(1, DH))

    eb = 8000
    gride = E // eb
    ce_proj, re_proj = pl.pallas_call(
        _tc1b_body,
        grid=(gride,),
        in_specs=[_node_block(eb, DE), _full_block((DE, DH)),
                  _full_block((DE, EOUT)), _full_block((1, EOUT))],
        out_specs=[pl.BlockSpec((NC, eb, DH2), lambda i: (0, i, 0)),
                   _node_block(eb, EOUT)],
        out_shape=[jax.ShapeDtypeStruct((NC, E, DH2), jnp.float32),
                   jax.ShapeDtypeStruct((E, EOUT), jnp.float32)],
    )(e, wme_t, wee_t, W_edge_b.reshape(1, EOUT))

    part1, deg_parts = _sc1(a_proj, b_proj, ce_proj, src, dst)

    q_proj, g_node, invd = pl.pallas_call(
        _tc2_body,
        grid=(gridn,),
        in_specs=[pl.BlockSpec((NC, nb, DH2), lambda i: (0, i, 0)),
                  pl.BlockSpec((NW, nb), lambda i: (0, i)),
                  _node_block(nb, D), _full_block((DH, EOUT)),
                  _full_block((D, NOUT)), _full_block((DH, NOUT)),
                  _full_block((1, NOUT))],
        out_specs=[_node_block(nb, EOUT), _node_block(nb, NOUT),
                   _node_block(nb, EOUT)],
        out_shape=[jax.ShapeDtypeStruct((N, EOUT), jnp.float32),
                   jax.ShapeDtypeStruct((N, NOUT), jnp.float32),
                   jax.ShapeDtypeStruct((N, EOUT), jnp.float32)],
    )(part1, deg_parts, x_dst, weh_t, wad_t, wah_t,
      W_apply_b.reshape(1, NOUT))

    eh, part2 = _sc2(p_proj, q_proj, re_proj, src, dst)

    h_out = pl.pallas_call(
        _tc3_body,
        grid=(gridn,),
        in_specs=[pl.BlockSpec((NC, nb, EOUT), lambda i: (0, i, 0)),
                  _node_block(nb, NOUT), _node_block(nb, EOUT),
                  _full_block((EOUT, NOUT))],
        out_specs=_node_block(nb, NOUT),
        out_shape=jax.ShapeDtypeStruct((N, NOUT), jnp.float32),
    )(part2, g_node, invd, wae_t)

    return (h_out, eh)


# speedup vs baseline: 4.0854x; 1.2084x over previous
import functools

import jax
import jax.numpy as jnp
from jax import lax
from jax.experimental import pallas as pl
from jax.experimental.pallas import tpu as pltpu
from jax.experimental.pallas import tpu_sc as plsc

N = 10000
E = 320000
D = 128
DE = 16
DH = 128
NOUT = 128
EOUT = 16

NC = 2
NS = 16
NW = NC * NS
EPT = E // NS
C1 = 80
NCH1 = EPT // C1
EPW = E // NW
C2 = 80
NCH2 = EPW // C2
NP = 10240
RPT = NP // NS
DH2 = DH // 2
E2 = E // 2
E8 = E // 8

_HIGH = jax.lax.Precision.HIGHEST


def _tc1a_body(xs_ref, xd_ref, wms_ref, wmd_ref, wes_ref, bm_ref,
               a_ref, b_ref, p_ref):
    xs = xs_ref[...]
    xd = xd_ref[...]
    a = jnp.dot(xs, wms_ref[...], precision=_HIGH) + bm_ref[...]
    b = jnp.dot(xd, wmd_ref[...], precision=_HIGH)
    a_ref[0] = a[:, 0:DH2]
    a_ref[1] = a[:, DH2:DH]
    b_ref[0] = b[:, 0:DH2]
    b_ref[1] = b[:, DH2:DH]
    p_ref[...] = jnp.dot(xs, wes_ref[...], precision=_HIGH)


def _tc1b_body(e2_ref, e8_ref, w2_ref, w8_ref, b8_ref, ce_ref, re_ref):
    e2 = e2_ref[...]
    ce_ref[0] = jnp.dot(e2, w2_ref[0], precision=_HIGH)
    ce_ref[1] = jnp.dot(e2, w2_ref[1], precision=_HIGH)
    re_ref[...] = jnp.dot(e8_ref[...], w8_ref[...],
                          precision=_HIGH) + b8_ref[...]


def _tc2_body(part_ref, degp_ref, xd_ref, weh_ref, wad_ref, wah_ref, ba_ref,
              q_ref, g_ref, invd_ref):
    hs = jnp.concatenate([part_ref[0], part_ref[1]], axis=1)
    deg = 0.5 * jnp.sum(degp_ref[...], axis=0)[:, None]
    denom = jnp.maximum(deg, 1.0)
    hn = hs / denom
    mask = (deg > 0.0).astype(jnp.float32)
    q_ref[...] = jnp.dot(hn, weh_ref[...], precision=_HIGH)
    g_ref[...] = mask * (jnp.dot(xd_ref[...], wad_ref[...], precision=_HIGH)
                         + jnp.dot(hn, wah_ref[...], precision=_HIGH)
                         + ba_ref[...])
    invd_ref[...] = jnp.broadcast_to(1.0 / denom, invd_ref.shape)


def _tc3_body(sp_ref, g_ref, invd_ref, wae_ref, out_ref):
    s = sp_ref[0] + sp_ref[1]
    out_ref[...] = g_ref[...] + (jnp.dot(s, wae_ref[...], precision=_HIGH)
                                 * invd_ref[:, 0:1])


def _node_block(nb, w):
    return pl.BlockSpec((nb, w), lambda i: (i, 0))


def _full_block(shape):
    return pl.BlockSpec(shape, lambda i: tuple(0 for _ in shape))


def _sc1_body(a_hbm, b_hbm, ce_hbm, src_hbm, dst_hbm, out_hbm, deg_hbm,
              sidx1, didx2, abuf, bbuf, cbuf, mbuf, degbuf, acc,
              sg0, sg1, si0, si1):
    cid = lax.axis_index("c")
    sid = lax.axis_index("s")
    wid = sid * NC + cid
    sgs = (sg0, sg1)
    sis = (si0, si1)

    ebase = sid * EPT

    def gathers(k, p):
        pltpu.async_copy(a_hbm.at[cid].at[sidx1.at[pl.ds(k * C1, C1)]],
                         abuf.at[p], sgs[p])
        pltpu.async_copy(b_hbm.at[cid].at[didx2.at[p]], bbuf.at[p], sgs[p])
        pltpu.async_copy(
            ce_hbm.at[cid].at[pl.ds(sid * (EPT // 2) + k * (C1 // 2),
                                    C1 // 2)],
            cbuf.at[p], sgs[p])

    pltpu.sync_copy(src_hbm.at[pl.ds(ebase, EPT)], sidx1)
    pltpu.sync_copy(dst_hbm.at[pl.ds(ebase, C1)], didx2.at[0])
    gathers(0, 0)
    pltpu.async_copy(dst_hbm.at[pl.ds(ebase + C1, C1)], didx2.at[1], si1)

    def zrow(r, _):
        for g in range(DH2 // 16):
            mbuf[r, pl.ds(g * 16, 16)] = jnp.zeros((16,), jnp.float32)
        return 0
    lax.fori_loop(0, C1, zrow, 0)
    base_r = sid * RPT
    for j in range(RPT // C1):
        pltpu.sync_copy(mbuf, acc.at[pl.ds(base_r + j * C1, C1)])

    def zdeg(i, _):
        degbuf[pl.ds(i * 16, 16)] = jnp.zeros((16,), jnp.float32)
        return 0
    lax.fori_loop(0, NP // 16, zdeg, 0)

    plsc.subcore_barrier()

    ones = jnp.full((16,), 1.0, jnp.float32)

    def chunk_body(k, p):
        pn = (p + 1) % 2
        pltpu.make_async_copy(a_hbm.at[cid].at[sidx1.at[pl.ds(k * C1, C1)]],
                              abuf.at[p], sgs[p]).wait()
        pltpu.make_async_copy(b_hbm.at[cid].at[didx2.at[p]], bbuf.at[p],
                              sgs[p]).wait()
        pltpu.make_async_copy(
            ce_hbm.at[cid].at[pl.ds(sid * (EPT // 2) + k * (C1 // 2),
                                    C1 // 2)],
            cbuf.at[p], sgs[p]).wait()

        @pl.when(k + 1 < NCH1)
        def _():
            pltpu.make_async_copy(dst_hbm.at[pl.ds(ebase, C1)],
                                  didx2.at[pn], sis[pn]).wait()
            gathers(k + 1, pn)

        def row(r2, _):
            for s in range(2):
                r = 2 * r2 + s
                for g in range(DH2 // 16):
                    sl = pl.ds(g * 16, 16)
                    v = (abuf[p, r, sl] + bbuf[p, r, sl]
                         + cbuf[p, r2, pl.ds(s * DH2 + g * 16, 16)])
                    mbuf[r, sl] = jnp.maximum(v, jnp.float32(0.0))
            return 0
        lax.fori_loop(0, C1 // 2, row, 0)

        def dgrp(g, _):
            idxv = didx2[p, pl.ds(g * 16, 16)]
            plsc.addupdate_scatter(degbuf, [idxv], ones)
            return 0
        lax.fori_loop(0, C1 // 16, dgrp, 0)

        pltpu.sync_copy(mbuf, acc.at[didx2.at[p]], add=True)

        @pl.when(k + 2 < NCH1)
        def _():
            pltpu.async_copy(dst_hbm.at[pl.ds(ebase + (k + 2) * C1, C1)],
                             didx2.at[p], sis[p])

    def step(k2, _):
        chunk_body(2 * k2, 0)
        chunk_body(2 * k2 + 1, 1)
        return 0

    lax.fori_loop(0, NCH1 // 2, step, 0)
    plsc.subcore_barrier()

    pltpu.sync_copy(acc.at[pl.ds(base_r, RPT)],
                    out_hbm.at[cid, pl.ds(base_r, RPT)])
    pltpu.sync_copy(degbuf, deg_hbm.at[wid])


def _sc2_body(p_hbm, q_hbm, re_hbm, src_hbm, dst_hbm, eh_hbm, out_hbm,
              sidx1, didx2, pbuf, qbuf, rbuf, ebuf, acc,
              sg0, sg1, si0, si1):
    cid = lax.axis_index("c")
    sid = lax.axis_index("s")
    wid = sid * NC + cid
    sgs = (sg0, sg1)
    sis = (si0, si1)

    ebase = wid * EPW

    def gathers(k, p):
        pltpu.async_copy(p_hbm.at[sidx1.at[pl.ds(k * C2, C2)]],
                         pbuf.at[p], sgs[p])
        pltpu.async_copy(q_hbm.at[didx2.at[p]], qbuf.at[p], sgs[p])
        pltpu.async_copy(
            re_hbm.at[pl.ds(wid * (EPW // 8) + k * (C2 // 8), C2 // 8)],
            rbuf.at[p], sgs[p])

    pltpu.sync_copy(src_hbm.at[pl.ds(ebase, EPW)], sidx1)
    pltpu.sync_copy(dst_hbm.at[pl.ds(ebase, C2)], didx2.at[0])
    gathers(0, 0)
    pltpu.async_copy(dst_hbm.at[pl.ds(ebase + C2, C2)], didx2.at[1], si1)

    def zrow(r, _):
        ebuf[r, pl.ds(0, 16)] = jnp.zeros((16,), jnp.float32)
        return 0
    lax.fori_loop(0, C2, zrow, 0)
    base_r = sid * RPT
    for j in range(RPT // C2):
        pltpu.sync_copy(ebuf, acc.at[pl.ds(base_r + j * C2, C2)])

    plsc.subcore_barrier()

    def chunk_body(k, p):
        pn = (p + 1) % 2
        pltpu.make_async_copy(p_hbm.at[sidx1.at[pl.ds(k * C2, C2)]],
                              pbuf.at[p], sgs[p]).wait()
        pltpu.make_async_copy(q_hbm.at[didx2.at[p]], qbuf.at[p],
                              sgs[p]).wait()
        pltpu.make_async_copy(
            re_hbm.at[pl.ds(wid * (EPW // 8) + k * (C2 // 8), C2 // 8)],
            rbuf.at[p], sgs[p]).wait()

        @pl.when(k + 1 < NCH2)
        def _():
            pltpu.make_async_copy(dst_hbm.at[pl.ds(ebase, C2)],
                                  didx2.at[pn], sis[pn]).wait()
            gathers(k + 1, pn)

        def row(r2, _):
            sl = pl.ds(0, 16)
            for s in range(8):
                r = 8 * r2 + s
                v = (pbuf[p, r, sl] + qbuf[p, r, sl]
                     + rbuf[p, r2, pl.ds(s * 16, 16)])
                ebuf[r, sl] = jnp.maximum(v, jnp.float32(0.0))
            return 0
        lax.fori_loop(0, C2 // 8, row, 0)

        pltpu.sync_copy(ebuf, eh_hbm.at[pl.ds(ebase + k * C2, C2)])
        pltpu.sync_copy(ebuf, acc.at[didx2.at[p]], add=True)

        @pl.when(k + 2 < NCH2)
        def _():
            pltpu.async_copy(dst_hbm.at[pl.ds(ebase + (k + 2) * C2, C2)],
                             didx2.at[p], sis[p])

    def step(k2, _):
        chunk_body(2 * k2, 0)
        chunk_body(2 * k2 + 1, 1)
        return 0

    lax.fori_loop(0, NCH2 // 2, step, 0)
    chunk_body(NCH2 - 1, 0)
    plsc.subcore_barrier()

    pltpu.sync_copy(acc.at[pl.ds(base_r, RPT)],
                    out_hbm.at[cid, pl.ds(base_r, RPT)])


_sc_mesh = plsc.VectorSubcoreMesh(core_axis_name="c", subcore_axis_name="s",
                                  num_cores=NC, num_subcores=NS)

_sc1 = functools.partial(
    pl.kernel, _sc1_body,
    out_type=[jax.ShapeDtypeStruct((NC, NP, DH2), jnp.float32),
              jax.ShapeDtypeStruct((NW, NP), jnp.float32)],
    mesh=_sc_mesh,
    compiler_params=pltpu.CompilerParams(use_tc_tiling_on_sc=False,
                                         needs_layout_passes=False),
    scratch_types=[
        pltpu.VMEM((EPT,), jnp.int32),
        pltpu.VMEM((2, C1), jnp.int32),
        pltpu.VMEM((2, C1, DH2), jnp.float32),
        pltpu.VMEM((2, C1, DH2), jnp.float32),
        pltpu.VMEM((2, C1 // 2, DH), jnp.float32),
        pltpu.VMEM((C1, DH2), jnp.float32),
        pltpu.VMEM((NP,), jnp.float32),
        pltpu.VMEM_SHARED((NP, DH2), jnp.float32),
        pltpu.SemaphoreType.DMA,
        pltpu.SemaphoreType.DMA,
        pltpu.SemaphoreType.DMA,
        pltpu.SemaphoreType.DMA,
    ],
)()

_sc2 = functools.partial(
    pl.kernel, _sc2_body,
    out_type=[jax.ShapeDtypeStruct((E, EOUT), jnp.float32),
              jax.ShapeDtypeStruct((NC, NP, EOUT), jnp.float32)],
    mesh=_sc_mesh,
    compiler_params=pltpu.CompilerParams(use_tc_tiling_on_sc=False,
                                         needs_layout_passes=False),
    scratch_types=[
        pltpu.VMEM((EPW,), jnp.int32),
        pltpu.VMEM((2, C2), jnp.int32),
        pltpu.VMEM((2, C2, EOUT), jnp.float32),
        pltpu.VMEM((2, C2, EOUT), jnp.float32),
        pltpu.VMEM((2, C2 // 8, DH), jnp.float32),
        pltpu.VMEM((C2, EOUT), jnp.float32),
        pltpu.VMEM_SHARED((NP, EOUT), jnp.float32),
        pltpu.SemaphoreType.DMA,
        pltpu.SemaphoreType.DMA,
        pltpu.SemaphoreType.DMA,
        pltpu.SemaphoreType.DMA,
    ],
)()


def kernel(x_src, x_dst, e, W_msg_w, W_msg_b, W_edge_w, W_edge_b,
           W_apply_w, W_apply_b, edge_index):
    src = edge_index[0]
    dst = edge_index[1]

    wms_t = W_msg_w[:, 0:D].T
    wme_t = W_msg_w[:, D:D + DE].T
    wmd_t = W_msg_w[:, D + DE:].T
    wes_t = W_edge_w[:, 0:D].T
    wee_t = W_edge_w[:, D:D + DE].T
    weh_t = W_edge_w[:, D + DE:].T
    wad_t = W_apply_w[:, 0:D].T
    wah_t = W_apply_w[:, D:D + DH].T
    wae_t = W_apply_w[:, D + DH:].T

    nb = 2048
    gridn = pl.cdiv(N, nb)

    a_proj, b_proj, p_proj = pl.pallas_call(
        _tc1a_body,
        grid=(gridn,),
        in_specs=[_node_block(nb, D), _node_block(nb, D),
                  _full_block((D, DH)), _full_block((D, DH)),
                  _full_block((D, EOUT)), _full_block((1, DH))],
        out_specs=[pl.BlockSpec((NC, nb, DH2), lambda i: (0, i, 0)),
                   pl.BlockSpec((NC, nb, DH2), lambda i: (0, i, 0)),
                   _node_block(nb, EOUT)],
        out_shape=[jax.ShapeDtypeStruct((NC, N, DH2), jnp.float32),
                   jax.ShapeDtypeStruct((NC, N, DH2), jnp.float32),
                   jax.ShapeDtypeStruct((N, EOUT), jnp.float32)],
    )(x_src, x_dst, wms_t, wmd_t, wes_t, W_msg_b.reshape(1, DH))

    w2 = jnp.zeros((NC, 2 * DE, DH), jnp.float32)
    for c in range(NC):
        half = wme_t[:, c * DH2:(c + 1) * DH2]
        w2 = w2.at[c, 0:DE, 0:DH2].set(half)
        w2 = w2.at[c, DE:2 * DE, DH2:DH].set(half)
    w8 = jnp.zeros((8 * DE, 8 * EOUT), jnp.float32)
    for s in range(8):
        w8 = w8.at[s * DE:(s + 1) * DE, s * EOUT:(s + 1) * EOUT].set(wee_t)
    b8 = jnp.tile(W_edge_b, 8).reshape(1, 8 * EOUT)

    e2 = e.reshape(E2, 2 * DE)
    e8 = e.reshape(E8, 8 * DE)

    gride = 20
    eb2 = E2 // gride
    eb8 = E8 // gride
    ce_proj, re_proj = pl.pallas_call(
        _tc1b_body,
        grid=(gride,),
        in_specs=[_node_block(eb2, 2 * DE), _node_block(eb8, 8 * DE),
                  _full_block((NC, 2 * DE, DH)),
                  _full_block((8 * DE, 8 * EOUT)),
                  _full_block((1, 8 * EOUT))],
        out_specs=[pl.BlockSpec((NC, eb2, DH), lambda i: (0, i, 0)),
                   _node_block(eb8, 8 * EOUT)],
        out_shape=[jax.ShapeDtypeStruct((NC, E2, DH), jnp.float32),
                   jax.ShapeDtypeStruct((E8, 8 * EOUT), jnp.float32)],
    )(e2, e8, w2, w8, b8)

    part1, deg_parts = _sc1(a_proj, b_proj, ce_proj, src, dst)

    q_proj, g_node, invd = pl.pallas_call(
        _tc2_body,
        grid=(gridn,),
        in_specs=[pl.BlockSpec((NC, nb, DH2), lambda i: (0, i, 0)),
                  pl.BlockSpec((NW, nb), lambda i: (0, i)),
                  _node_block(nb, D), _full_block((DH, EOUT)),
                  _full_block((D, NOUT)), _full_block((DH, NOUT)),
                  _full_block((1, NOUT))],
        out_specs=[_node_block(nb, EOUT), _node_block(nb, NOUT),
                   _node_block(nb, EOUT)],
        out_shape=[jax.ShapeDtypeStruct((N, EOUT), jnp.float32),
                   jax.ShapeDtypeStruct((N, NOUT), jnp.float32),
                   jax.ShapeDtypeStruct((N, EOUT), jnp.float32)],
    )(part1, deg_parts, x_dst, weh_t, wad_t, wah_t,
      W_apply_b.reshape(1, NOUT))

    eh, part2 = _sc2(p_proj, q_proj, re_proj, src, dst)

    h_out = pl.pallas_call(
        _tc3_body,
        grid=(gridn,),
        in_specs=[pl.BlockSpec((NC, nb, EOUT), lambda i: (0, i, 0)),
                  _node_block(nb, NOUT), _node_block(nb, EOUT),
                  _full_block((EOUT, NOUT))],
        out_specs=_node_block(nb, NOUT),
        out_shape=jax.ShapeDtypeStruct((N, NOUT), jnp.float32),
    )(part2, g_node, invd, wae_t)

    return (h_out, eh)
